# baseline (speedup 1.0000x reference)
import functools

import jax
import jax.numpy as jnp
from jax.experimental import pallas as pl
from jax.experimental.pallas import tpu as pltpu

_B, _F, _N, _E = 8, 1024, 2048, 128
_NEG = -1e30


def _hi_lo(x):
    hi = x.astype(jnp.bfloat16)
    lo = (x - hi.astype(jnp.float32)).astype(jnp.bfloat16)
    return hi, lo


def _body(rel_ref, arg1_ref, fr_ref, fa1_ref, fa2_ref, nbf_ref, ents_ref,
          nbe_ref, w0_ref, wp0_ref, w1_ref, wp1_ref, out_ref):
    rel = rel_ref[0]
    src = arg1_ref[0]
    fr = fr_ref[0]
    fa1 = fa1_ref[0]
    fa2 = fa2_ref[0]
    ents = ents_ref[0]
    nbf = nbf_ref[0, 0, 0]
    nbe = nbe_ref[0, 0, 0]

    inv_e = 1.0 / _E
    half = 0.5 * inv_e

    n_iota = jax.lax.broadcasted_iota(jnp.int32, (_N, 1), 0)
    xn = jnp.sum(ents * ents, axis=1, keepdims=True)
    xcol = jnp.where(n_iota < nbe, xn * -half, _NEG)
    x_hi, x_lo = _hi_lo(xcol)
    ones_n = jnp.ones((_N, 1), jnp.bfloat16)
    lhs = jnp.concatenate([ents.astype(jnp.bfloat16), x_hi, x_lo,
                           ones_n, ones_n], axis=1)

    f_iota = jax.lax.broadcasted_iota(jnp.int32, (_F, 1), 0)
    ones_f = jnp.ones((_F, 1), jnp.bfloat16)

    def one(w_ref, wp_ref, fa_src, fa_ent):
        hop = jnp.dot(rel, w_ref[...], preferred_element_type=jnp.float32)
        e1 = hop - fr
        e2 = src - fa_src
        q = e1 * e1 + e2 * e2 + fa_ent * fa_ent
        d = jnp.sum(q, axis=1, keepdims=True)
        c = jnp.where(f_iota < nbf, d * -half, _NEG)
        c_hi, c_lo = _hi_lo(c)
        rhs = jnp.concatenate([(fa_ent * inv_e).astype(jnp.bfloat16),
                               ones_f, ones_f, c_hi, c_lo], axis=1)
        xy = jax.lax.dot_general(lhs, rhs, (((1,), (1,)), ((), ())),
                                 preferred_element_type=jnp.float32)
        m = jnp.max(xy, axis=(0, 1), keepdims=True)
        logit = jnp.sum(rel * wp_ref[...], axis=(0, 1), keepdims=True)
        prior = jax.nn.sigmoid(logit)
        return jnp.minimum(jnp.exp(m), prior)

    r0 = one(w0_ref, wp0_ref, fa1, fa2)
    r1 = one(w1_ref, wp1_ref, fa2, fa1)
    out_ref[0] = jnp.broadcast_to(jnp.maximum(r0, r1), (1, _E))


@jax.jit
def kernel(rel, arg1, arg2, fact_rel, fact_arg1, fact_arg2, nb_facts,
           entity_embeddings, nb_entities, W_hop_0, w_prior_0, W_hop_1,
           w_prior_1):
    del arg2
    nbf = nb_facts.reshape(_B, 1, 1)
    nbe = nb_entities.reshape(_B, 1, 1)
    wp0 = w_prior_0.reshape(1, _E)
    wp1 = w_prior_1.reshape(1, _E)
    rel3 = rel.reshape(_B, 1, _E)
    arg13 = arg1.reshape(_B, 1, _E)

    vec = pl.BlockSpec((1, 1, _E), lambda b: (b, 0, 0))
    facts = pl.BlockSpec((1, _F, _E), lambda b: (b, 0, 0))
    smem = pl.BlockSpec((1, 1, 1), lambda b: (b, 0, 0),
                        memory_space=pltpu.SMEM)
    const2 = pl.BlockSpec((_E, _E), lambda b: (0, 0))
    const_row = pl.BlockSpec((1, _E), lambda b: (0, 0))

    out = pl.pallas_call(
        _body,
        grid=(_B,),
        in_specs=[vec, vec, facts, facts, facts, smem,
                  pl.BlockSpec((1, _N, _E), lambda b: (b, 0, 0)), smem,
                  const2, const_row, const2, const_row],
        out_specs=pl.BlockSpec((1, 1, _E), lambda b: (b, 0, 0)),
        out_shape=jax.ShapeDtypeStruct((_B, 1, _E), jnp.float32),
        compiler_params=pltpu.CompilerParams(
            dimension_semantics=("parallel",)),
    )(rel3, arg13, fact_rel, fact_arg1, fact_arg2, nbf, entity_embeddings,
      nbe, W_hop_0, wp0, W_hop_1, wp1)
    return out[:, 0, 0]
